# Initial kernel scaffold; baseline (speedup 1.0000x reference)
import jax
import jax.numpy as jnp
from jax.experimental import pallas as pl


def kernel(feat, batch, lengths, sorter_scores, Wq, bq, Wk, bk, Wv, bv, Wg, bg, Wp, bp, W1, b1f, W2, b2f, g1, be1, g2, be2, gbn, bbn):
    raise NotImplementedError("write your pallas kernel here")


# speedup vs baseline: 1.3162x; 1.3162x over previous
import jax
import jax.numpy as jnp
from jax import lax
from jax.experimental import pallas as pl
from jax.experimental.pallas import tpu as pltpu
from jax.experimental.pallas import tpu_sc as plsc

N = 160000
C = 128
H = 4
B = 16
HID = 256
WIN = 128
TOTAL = B * ((40000 + WIN - 1) // WIN)
TOTAL_PAD = 5120
SCALE = (C // H) ** -0.5

GROUPS = N // 128
NW = 32
NPAD = 163840
GPADS = NPAD // 128
NPASS = 3
HALF = 1792
HTAB = HALF + 8
TROWS = HALF // 16
MROWS = HALF // 16
MTOT = NPASS * HALF
SAFE_ROW = 5100

BLK = 640
NBLK = N // BLK
GROWS = BLK // 128


def _ln(x, g, b):
    mu = jnp.mean(x, axis=-1, keepdims=True)
    var = jnp.mean((x - mu) ** 2, axis=-1, keepdims=True)
    return (x - mu) * jax.lax.rsqrt(var + 1e-5) * g + b


def _qkv_body(feat_ref, batch_ref, score_ref, nb_ref, boff_ref,
              wq_ref, bq_ref, wk_ref, bk_ref, wv_ref, bv_ref,
              g1_ref, be1_ref, q_ref, kv_ref, gb_ref, h0_ref, h1_ref):
    x = _ln(feat_ref[...], g1_ref[...], be1_ref[...])
    q_ref[...] = jnp.dot(x, wq_ref[...], preferred_element_type=jnp.float32) + bq_ref[...]
    kv_ref[0] = jnp.dot(x, wk_ref[...], preferred_element_type=jnp.float32) + bk_ref[...]
    kv_ref[1] = jnp.dot(x, wv_ref[...], preferred_element_type=jnp.float32) + bv_ref[...]
    b = batch_ref[0]
    s = score_ref[0]
    nbp = jnp.zeros_like(b)
    bop = jnp.zeros_like(b)
    for j in range(B):
        nbp = jnp.where(b == j, nb_ref[0, j], nbp)
        bop = jnp.where(b == j, boff_ref[0, j], bop)
    local = (s * nbp.astype(jnp.float32)).astype(jnp.int32)
    local = jnp.minimum(local, nbp - 1)
    gb = local + bop
    gb_ref[0] = gb
    h0_ref[0] = jnp.where(gb < HALF, gb, HALF)
    rel = gb - HALF
    h1_ref[0] = jnp.where(rel >= 0, rel, HALF)


def _qkv(feat, batch3d, scores3d, nb32, boff32, Wq, bq, Wk, bk, Wv, bv, g1, be1):
    wspec = pl.BlockSpec((C, C), lambda i: (0, 0))
    bspec = pl.BlockSpec((1, C), lambda i: (0, 0))
    i16spec = pl.BlockSpec((1, B), lambda i: (0, 0))
    blk = pl.BlockSpec((BLK, C), lambda i: (i, 0))
    kvblk = pl.BlockSpec((2, BLK, C), lambda i: (0, i, 0))
    gblk = pl.BlockSpec((1, GROWS, 128), lambda i: (i, 0, 0))
    return pl.pallas_call(
        _qkv_body,
        grid=(NBLK,),
        in_specs=[blk, gblk, gblk, i16spec, i16spec,
                  wspec, bspec, wspec, bspec, wspec, bspec, bspec, bspec],
        out_specs=[blk, kvblk, gblk, gblk, gblk],
        out_shape=[jax.ShapeDtypeStruct((N, C), jnp.float32),
                   jax.ShapeDtypeStruct((2, N, C), jnp.float32)]
        + [jax.ShapeDtypeStruct((NBLK, GROWS, 128), jnp.int32)] * 3,
    )(feat, batch3d, scores3d, nb32.reshape(1, B), boff32.reshape(1, B),
      Wq, bq.reshape(1, C), Wk, bk.reshape(1, C), Wv, bv.reshape(1, C),
      g1.reshape(1, C), be1.reshape(1, C))


GP_TILE = GPADS // 16
GP_WORK = GPADS // NW
INNER = 8


def _sc_body(kv_hbm, gb_hbm, mean_out, ctx_out,
             idx_v, idx2_v, dbuf, gbuf, ones_v, zinit, czinit,
             zbuf, czbuf, sem, tab, ctab):
    cid = lax.axis_index("c")
    sid = lax.axis_index("s")
    wid = sid * 2 + cid

    zv = jnp.zeros((16,), jnp.float32)
    for r in range(8):
        czinit[r, :] = zv
        for c in range(8):
            zinit[r, pl.ds(c * 16, 16)] = zv
    ov = jnp.ones((16,), jnp.float32)
    for r in range(128):
        ones_v[r, :] = ov

    for p in range(NPASS):
        def zero_body(jj, carry):
            pltpu.sync_copy(zinit, tab.at[pl.ds(sid * TROWS + jj * 8, 8)])
            return carry

        def zero_cnt(jj, carry):
            pltpu.sync_copy(czinit, ctab.at[pl.ds(sid * TROWS + jj * 8, 8)])
            return carry

        lax.fori_loop(0, TROWS // 8, zero_body, 0)
        lax.fori_loop(0, TROWS // 8, zero_cnt, 0)
        pltpu.sync_copy(zinit, tab.at[pl.ds(HALF, 8)])
        pltpu.sync_copy(czinit, ctab.at[pl.ds(HALF, 8)])
        plsc.subcore_barrier()

        def scat_body(jo, carry, _p=p):
            for t in range(INNER):
                g = sid * GP_TILE + jo * INNER + t
                off = g * 128
                pltpu.sync_copy(gb_hbm.at[pl.ds(off, 128)], idx_v)
                for c in range(8):
                    gbc = idx_v[pl.ds(c * 16, 16)]
                    if _p == 0:
                        fc = jnp.minimum(gbc, HALF)
                    else:
                        rel = gbc - _p * HALF
                        fc = jnp.where(rel >= 0, jnp.minimum(rel, HALF), HALF)
                    idx2_v[pl.ds(c * 16, 16)] = fc
                pltpu.sync_copy(kv_hbm.at[cid, pl.ds(off, 128)], dbuf)
                pltpu.sync_copy(dbuf, tab.at[idx2_v], add=True)
                pltpu.sync_copy(ones_v, ctab.at[idx2_v], add=True)
            return carry

        lax.fori_loop(0, GP_TILE // INNER, scat_body, 0)
        plsc.subcore_barrier()

        m0 = sid * MROWS
        pltpu.sync_copy(tab.at[pl.ds(m0, MROWS)], zbuf)
        pltpu.sync_copy(ctab.at[pl.ds(m0, MROWS)], czbuf)

        def div_body(r, carry):
            inv = 1.0 / jnp.maximum(czbuf[r, :], 1.0)
            for c in range(8):
                zbuf[r, pl.ds(c * 16, 16)] = zbuf[r, pl.ds(c * 16, 16)] * inv
            return carry

        lax.fori_loop(0, MROWS, div_body, 0)
        pltpu.sync_copy(zbuf, mean_out.at[pl.ds(p * HALF + m0, MROWS),
                                          pl.ds(cid * C, C)])
        plsc.subcore_barrier()

    def gath_body(jo, carry):
        for t in range(INNER):
            g = wid * GP_WORK + jo * INNER + t
            off = g * 128
            pltpu.sync_copy(gb_hbm.at[pl.ds(off, 128)], idx_v)
            pltpu.async_copy(mean_out.at[idx_v], gbuf, sem).wait()
            pltpu.sync_copy(gbuf, ctx_out.at[pl.ds(off, 128)])
        return carry

    lax.fori_loop(0, GP_WORK // INNER, gath_body, 0)


def _sc_segmean_gather(kv, gb_flat, h0_flat, h1_flat):
    mesh = plsc.VectorSubcoreMesh(core_axis_name="c", subcore_axis_name="s")
    pad = jnp.full((NPAD - N,), SAFE_ROW, jnp.int32)
    gb_pad = jnp.concatenate([gb_flat, pad])
    out_type = [
        jax.ShapeDtypeStruct((MTOT, 2 * C), jnp.float32),
        jax.ShapeDtypeStruct((NPAD, 2 * C), jnp.float32),
    ]
    scratch = [
        pltpu.VMEM((128,), jnp.int32),
        pltpu.VMEM((128,), jnp.int32),
        pltpu.VMEM((128, C), jnp.float32),
        pltpu.VMEM((128, 2 * C), jnp.float32),
        pltpu.VMEM((128, 16), jnp.float32),
        pltpu.VMEM((8, C), jnp.float32),
        pltpu.VMEM((8, 16), jnp.float32),
        pltpu.VMEM((MROWS, C), jnp.float32),
        pltpu.VMEM((MROWS, 16), jnp.float32),
        pltpu.SemaphoreType.DMA,
        pltpu.VMEM_SHARED((HTAB, C), jnp.float32),
        pltpu.VMEM_SHARED((HTAB, 16), jnp.float32),
    ]
    fn = pl.kernel(_sc_body, out_type=out_type, mesh=mesh,
                   scratch_types=scratch)
    _, ctx = fn(kv, gb_pad)
    return ctx


def _attn_body(feat_ref, q_ref, ctx_ref, wg_ref, bg_ref,
               wp_ref, bp_ref, g2_ref, be2_ref, x1_ref, s_ref, m_ref):
    i = pl.program_id(0)
    q = q_ref[...]
    ctx = ctx_ref[...]
    kctx = ctx[:, :C]
    vctx = ctx[:, C:]
    z = jnp.dot(q * kctx * SCALE, wg_ref[...],
                preferred_element_type=jnp.float32) + bg_ref[...]
    attn = jax.nn.sigmoid(z)
    xo = jnp.dot(attn * vctx, wp_ref[...],
                 preferred_element_type=jnp.float32) + bp_ref[...]
    x1 = feat_ref[...] + xo
    x1_ref[...] = x1
    h = _ln(x1, g2_ref[...], be2_ref[...])

    @pl.when(i == 0)
    def _():
        s_ref[...] = jnp.zeros_like(s_ref)
        m_ref[...] = jnp.zeros_like(m_ref)

    s_ref[...] += lax.dot_general(h, h, (((0,), (0,)), ((), ())),
                                  preferred_element_type=jnp.float32)
    colsum = jnp.sum(h, axis=0, keepdims=True)
    rows = lax.broadcasted_iota(jnp.int32, (8, C), 0)
    m_ref[...] += jnp.where(rows == 0, jnp.broadcast_to(colsum, (8, C)), 0.0)


def _attn_apply(feat, q, ctx, Wg, bg, Wp, bp, g2, be2):
    wspec = pl.BlockSpec((C, C), lambda i: (0, 0))
    bspec = pl.BlockSpec((1, C), lambda i: (0, 0))
    blk = pl.BlockSpec((BLK, C), lambda i: (i, 0))
    cblk = pl.BlockSpec((BLK, 2 * C), lambda i: (i, 0))
    return pl.pallas_call(
        _attn_body,
        grid=(NBLK,),
        in_specs=[blk, blk, cblk, wspec, bspec, wspec, bspec, bspec, bspec],
        out_specs=[blk,
                   pl.BlockSpec((C, C), lambda i: (0, 0)),
                   pl.BlockSpec((8, C), lambda i: (0, 0))],
        out_shape=[jax.ShapeDtypeStruct((N, C), jnp.float32),
                   jax.ShapeDtypeStruct((C, C), jnp.float32),
                   jax.ShapeDtypeStruct((8, C), jnp.float32)],
    )(feat, q, ctx, Wg, bg.reshape(1, C), Wp, bp.reshape(1, C),
      g2.reshape(1, C), be2.reshape(1, C))


def _stats_body(s_ref, m_ref, w1_ref, b1_ref, gbn_ref, bbn_ref,
                w1a_ref, b1a_ref):
    mean_h = jnp.sum(m_ref[...], axis=0, keepdims=True) / N
    m2w = jnp.dot(s_ref[...] / N, w1_ref[...],
                  preferred_element_type=jnp.float32)
    es = jnp.dot(mean_h, w1_ref[...], preferred_element_type=jnp.float32)
    b1 = b1_ref[...]
    mu = es + b1
    e2 = jnp.sum(w1_ref[...] * m2w, axis=0, keepdims=True) + 2.0 * b1 * es + b1 * b1
    var = e2 - mu * mu
    a = gbn_ref[...] * lax.rsqrt(var + 1e-5)
    shift = bbn_ref[...] - mu * a
    w1a_ref[...] = w1_ref[...] * a
    rows = lax.broadcasted_iota(jnp.int32, (8, HID), 0)
    b1a_ref[...] = jnp.where(rows == 0,
                             jnp.broadcast_to(b1 * a + shift, (8, HID)), 0.0)


def _stats(S, m8, W1, b1f, gbn, bbn):
    return pl.pallas_call(
        _stats_body,
        out_shape=[jax.ShapeDtypeStruct((C, HID), jnp.float32),
                   jax.ShapeDtypeStruct((8, HID), jnp.float32)],
    )(S, m8, W1, b1f.reshape(1, HID), gbn.reshape(1, HID), bbn.reshape(1, HID))


def _mlp_body(x1_ref, w1a_ref, b1a_ref, w2_ref, b2_ref, g2_ref, be2_ref,
              out_ref):
    x1 = x1_ref[...]
    h = _ln(x1, g2_ref[...], be2_ref[...])
    hr = jax.nn.relu(jnp.dot(h, w1a_ref[...],
                             preferred_element_type=jnp.float32) + b1a_ref[...])
    out_ref[...] = x1 + jnp.dot(hr, w2_ref[...],
                                preferred_element_type=jnp.float32) + b2_ref[...]


def _mlp(x1, W1a, b1a, W2, b2f, g2, be2):
    blk = pl.BlockSpec((BLK, C), lambda i: (i, 0))
    return pl.pallas_call(
        _mlp_body,
        grid=(NBLK,),
        in_specs=[blk,
                  pl.BlockSpec((C, HID), lambda i: (0, 0)),
                  pl.BlockSpec((1, HID), lambda i: (0, 0)),
                  pl.BlockSpec((HID, C), lambda i: (0, 0)),
                  pl.BlockSpec((1, C), lambda i: (0, 0)),
                  pl.BlockSpec((1, C), lambda i: (0, 0)),
                  pl.BlockSpec((1, C), lambda i: (0, 0))],
        out_specs=blk,
        out_shape=jax.ShapeDtypeStruct((N, C), jnp.float32),
    )(x1, W1a, b1a, W2, b2f.reshape(1, C), g2.reshape(1, C), be2.reshape(1, C))


def kernel(feat, batch, lengths, sorter_scores, Wq, bq, Wk, bk, Wv, bv,
           Wg, bg, Wp, bp, W1, b1f, W2, b2f, g1, be1, g2, be2, gbn, bbn):
    batch3d = batch.astype(jnp.int32).reshape(NBLK, GROWS, 128)
    scores3d = sorter_scores.reshape(NBLK, GROWS, 128)
    lengths32 = lengths.astype(jnp.int32)
    nb32 = (lengths32 + (WIN - 1)) // WIN
    boff32 = jnp.cumsum(nb32) - nb32

    q, kv, gb3d, h03d, h13d = _qkv(feat, batch3d, scores3d, nb32, boff32,
                                   Wq, bq, Wk, bk, Wv, bv, g1, be1)
    ctx = _sc_segmean_gather(kv, gb3d.reshape(N), h03d.reshape(N),
                             h13d.reshape(N))
    x1, S, m8 = _attn_apply(feat, q, ctx, Wg, bg, Wp, bp, g2, be2)
    W1a, b1a8 = _stats(S, m8, W1, b1f, gbn, bbn)
    out = _mlp(x1, W1a, b1a8[0:1], W2, b2f, g2, be2)
    return out
